# Initial kernel scaffold; baseline (speedup 1.0000x reference)
import jax
import jax.numpy as jnp
from jax.experimental import pallas as pl


def kernel(out1, out2, t):
    raise NotImplementedError("write your pallas kernel here")


# speedup vs baseline: 4.1593x; 4.1593x over previous
import functools

import jax
import jax.numpy as jnp
from jax.experimental import pallas as pl

_INTERPRET = False


def _topk_body(tc_ref, tr_ref, out_ref, *, T, B, R, N):
    i = pl.program_id(0)
    row_base = i * R
    trow = tc_ref[...]
    tall = tr_ref[...]
    diff = jnp.abs(trow - tall)
    col_iota = jax.lax.broadcasted_iota(jnp.int32, (R, N), 1)
    row_iota = jax.lax.broadcasted_iota(jnp.int32, (R, N), 0) + row_base
    same = (row_iota // T) == (col_iota // T)
    key = jnp.where(same, jnp.float32(1e6), diff)
    cols = []
    for _ in range(B):
        m = jnp.min(key, axis=1, keepdims=True)
        idx = jnp.min(
            jnp.where(key == m, col_iota, jnp.int32(2**30)),
            axis=1, keepdims=True)
        cols.append(idx)
        key = jnp.where(col_iota == idx, jnp.float32(jnp.inf), key)
    out_ref[...] = jnp.concatenate(cols, axis=1)


def _topk_pallas(tf, T, B):
    N = tf.shape[0]
    R = min(N, 256)
    t_col = tf.reshape(N, 1)
    t_row = tf.reshape(1, N)
    return pl.pallas_call(
        functools.partial(_topk_body, T=T, B=B, R=R, N=N),
        grid=(N // R,),
        in_specs=[
            pl.BlockSpec((R, 1), lambda i: (i, 0)),
            pl.BlockSpec((1, N), lambda i: (0, 0)),
        ],
        out_specs=pl.BlockSpec((R, B), lambda i: (i, 0)),
        out_shape=jax.ShapeDtypeStruct((N, B), jnp.int32),
        interpret=_INTERPRET,
    )(t_col, t_row)


def kernel(out1, out2, t):
    B, T, D = out1.shape
    z1, z2, tt = out1, out2, t.astype(jnp.float32)
    total = jnp.float32(0.0)
    d = 0
    while True:
        Tl = z1.shape[1]
        N = B * Tl
        tf = tt.reshape(N)
        idx = _topk_pallas(tf, Tl, B)
        z1f = z1.reshape(N, D)
        z2f = z2.reshape(N, D)
        pos = jnp.sum(z1f * z2f, axis=1)
        g1 = z1f[idx]
        g2 = z2f[idx]
        neg1 = jnp.einsum('nd,nkd->nk', z1f, g1)
        neg2 = jnp.einsum('nd,nkd->nk', z1f, g2)
        logits = jnp.concatenate([pos[:, None], neg1, neg2], axis=1)
        lse = jax.scipy.special.logsumexp(logits, axis=1)
        total = total + jnp.mean(lse - pos)
        d += 1
        if Tl == 1:
            break
        T2 = Tl // 2
        tt = tt.reshape(B, T2, 2).mean(axis=2)
        z1 = z1.reshape(B, T2, 2, D).max(axis=2)
        z2 = z2.reshape(B, T2, 2, D).max(axis=2)
    return total / d


# speedup vs baseline: 4.8727x; 1.1715x over previous
import functools

import jax
import jax.numpy as jnp
from jax.experimental import pallas as pl

_INTERPRET = False


def _topk_body(tc_ref, tr_ref, out_ref, *, T, B, R, N):
    i = pl.program_id(0)
    row_base = i * R
    trow = tc_ref[...]
    tall = tr_ref[...]
    diff = jnp.abs(trow - tall)
    col_iota = jax.lax.broadcasted_iota(jnp.int32, (R, N), 1)
    row_iota = jax.lax.broadcasted_iota(jnp.int32, (R, N), 0) + row_base
    same = (row_iota // T) == (col_iota // T)
    masked = jnp.where(same, jnp.float32(1e6), diff)
    key = (jax.lax.bitcast_convert_type(masked, jnp.int32) & ~0x1FFF) | col_iota
    cols = []
    for _ in range(B):
        m = jnp.min(key, axis=1, keepdims=True)
        cols.append(m & 0x1FFF)
        key = jnp.where(key == m, jnp.int32(0x7FFFFFFF), key)
    out_ref[...] = jnp.concatenate(cols, axis=1)


def _topk_pallas(tf, T, B):
    N = tf.shape[0]
    R = min(N, 256)
    t_col = tf.reshape(N, 1)
    t_row = tf.reshape(1, N)
    return pl.pallas_call(
        functools.partial(_topk_body, T=T, B=B, R=R, N=N),
        grid=(N // R,),
        in_specs=[
            pl.BlockSpec((R, 1), lambda i: (i, 0)),
            pl.BlockSpec((1, N), lambda i: (0, 0)),
        ],
        out_specs=pl.BlockSpec((R, B), lambda i: (i, 0)),
        out_shape=jax.ShapeDtypeStruct((N, B), jnp.int32),
        interpret=_INTERPRET,
    )(t_col, t_row)


def kernel(out1, out2, t):
    B, T, D = out1.shape
    z1, z2, tt = out1, out2, t.astype(jnp.float32)
    total = jnp.float32(0.0)
    d = 0
    while True:
        Tl = z1.shape[1]
        N = B * Tl
        tf = tt.reshape(N)
        idx = _topk_pallas(tf, Tl, B)
        z1f = z1.reshape(N, D)
        z2f = z2.reshape(N, D)
        pos = jnp.sum(z1f * z2f, axis=1)
        g1 = z1f[idx]
        g2 = z2f[idx]
        neg1 = jnp.einsum('nd,nkd->nk', z1f, g1)
        neg2 = jnp.einsum('nd,nkd->nk', z1f, g2)
        logits = jnp.concatenate([pos[:, None], neg1, neg2], axis=1)
        lse = jax.scipy.special.logsumexp(logits, axis=1)
        total = total + jnp.mean(lse - pos)
        d += 1
        if Tl == 1:
            break
        T2 = Tl // 2
        tt = tt.reshape(B, T2, 2).mean(axis=2)
        z1 = z1.reshape(B, T2, 2, D).max(axis=2)
        z2 = z2.reshape(B, T2, 2, D).max(axis=2)
    return total / d


# speedup vs baseline: 10.5236x; 2.1597x over previous
import functools

import jax
import jax.numpy as jnp
from jax import lax
from jax.experimental import pallas as pl
from jax.experimental.pallas import tpu as pltpu
from jax.experimental.pallas import tpu_sc as plsc

_INTERPRET = False


def _topk_body(tc_ref, tr_ref, out_ref, *, T, B, R, N):
    i = pl.program_id(0)
    row_base = i * R
    trow = tc_ref[...]
    tall = tr_ref[...]
    diff = jnp.abs(trow - tall)
    col_iota = jax.lax.broadcasted_iota(jnp.int32, (R, N), 1)
    row_iota = jax.lax.broadcasted_iota(jnp.int32, (R, N), 0) + row_base
    same = (row_iota // T) == (col_iota // T)
    masked = jnp.where(same, jnp.float32(1e6), diff)
    key = (jax.lax.bitcast_convert_type(masked, jnp.int32) & ~0x1FFF) | col_iota
    cols = []
    for _ in range(B):
        m = jnp.min(key, axis=1, keepdims=True)
        cols.append(m & 0x1FFF)
        key = jnp.where(key == m, jnp.int32(0x7FFFFFFF), key)
    out_ref[...] = jnp.concatenate(cols, axis=1)


def _topk_pallas(tf, T, B):
    N = tf.shape[0]
    R = min(N, 256)
    t_col = tf.reshape(N, 1)
    t_row = tf.reshape(1, N)
    return pl.pallas_call(
        functools.partial(_topk_body, T=T, B=B, R=R, N=N),
        grid=(N // R,),
        in_specs=[
            pl.BlockSpec((R, 1), lambda i: (i, 0)),
            pl.BlockSpec((1, N), lambda i: (0, 0)),
        ],
        out_specs=pl.BlockSpec((R, B), lambda i: (i, 0)),
        out_shape=jax.ShapeDtypeStruct((N, B), jnp.int32),
        interpret=_INTERPRET,
    )(t_col, t_row)


def _gather_dot_sc(z1f, z2f, idx):
    N, D = z1f.shape
    B = idx.shape[1]
    info = plsc.get_sparse_core_info()
    NC, NS = info.num_cores, info.num_subcores
    NW = NC * NS
    RW = max(N // NW, 1)
    ACT = N // RW
    C = min(RW, 32)
    CHUNKS = RW // C
    idx_flat = idx.reshape(N * B)
    mesh = plsc.VectorSubcoreMesh(core_axis_name="c", subcore_axis_name="s")

    @functools.partial(
        pl.kernel, mesh=mesh,
        out_type=jax.ShapeDtypeStruct((N, 2 * B), jnp.float32),
        scratch_types=[
            pltpu.VMEM((C * B,), jnp.int32),
            pltpu.VMEM((C, D), jnp.float32),
            pltpu.VMEM((C * B, D), jnp.float32),
            pltpu.VMEM((C * B, D), jnp.float32),
            pltpu.VMEM((C, 2 * B), jnp.float32),
            pltpu.SemaphoreType.DMA,
        ])
    def sc_kernel(z1_hbm, z2_hbm, idx_hbm, out_hbm,
                  idx_v, own_v, g1_v, g2_v, out_v, sem):
        wid = lax.axis_index("s") * NC + lax.axis_index("c")

        @pl.when(wid < ACT)
        def _():
            def chunk_body(c, carry):
                rowbase = wid * RW + c * C
                pltpu.sync_copy(idx_hbm.at[pl.ds(rowbase * B, C * B)], idx_v)
                pltpu.async_copy(z1_hbm.at[idx_v], g1_v, sem).wait()
                pltpu.async_copy(z2_hbm.at[idx_v], g2_v, sem).wait()
                pltpu.sync_copy(z1_hbm.at[pl.ds(rowbase, C)], own_v)
                lane = lax.iota(jnp.int32, 16)
                perms = [((lane + s) & 15).reshape(16, 1) for s in (1, 2, 4, 8)]
                _gdims = lax.GatherDimensionNumbers(
                    offset_dims=(), collapsed_slice_dims=(0,),
                    start_index_map=(0,))

                def _lanesum(v):
                    for p in perms:
                        v = v + lax.gather(
                            v, p, _gdims, (1,),
                            mode=lax.GatherScatterMode.PROMISE_IN_BOUNDS)
                    return v

                def row_body(r, carry2):
                    z1r = [own_v[r, pl.ds(i * 16, 16)] for i in range(D // 16)]
                    vals = jnp.zeros((16,), jnp.float32)
                    for k in range(B):
                        acc1 = z1r[0] * g1_v[r * B + k, pl.ds(0, 16)]
                        acc2 = z1r[0] * g2_v[r * B + k, pl.ds(0, 16)]
                        for i in range(1, D // 16):
                            acc1 = acc1 + z1r[i] * g1_v[r * B + k, pl.ds(i * 16, 16)]
                            acc2 = acc2 + z1r[i] * g2_v[r * B + k, pl.ds(i * 16, 16)]
                        vals = jnp.where(lane == k, _lanesum(acc1), vals)
                        vals = jnp.where(lane == (k + B), _lanesum(acc2), vals)
                    out_v[r, :] = vals
                    return carry2

                lax.fori_loop(0, C, row_body, 0)
                pltpu.sync_copy(out_v, out_hbm.at[pl.ds(rowbase, C)])
                return carry

            lax.fori_loop(0, CHUNKS, chunk_body, 0)

    return sc_kernel(z1f, z2f, idx_flat)


def _loss_body(z1_ref, z2_ref, neg_ref, out_ref):
    i = pl.program_id(0)
    z1 = z1_ref[...]
    z2 = z2_ref[...]
    neg = neg_ref[...]
    pos = jnp.sum(z1 * z2, axis=1, keepdims=True)
    m = jnp.maximum(jnp.max(neg, axis=1, keepdims=True), pos)
    s = jnp.sum(jnp.exp(neg - m), axis=1, keepdims=True) + jnp.exp(pos - m)
    lse = m + jnp.log(s)
    part = jnp.sum(lse - pos).reshape(1, 1)

    @pl.when(i == 0)
    def _():
        out_ref[...] = jnp.zeros((1, 1), jnp.float32)

    out_ref[...] += part


def _loss_pallas(z1f, z2f, neg):
    N, D = z1f.shape
    R = min(N, 256)
    return pl.pallas_call(
        _loss_body,
        grid=(N // R,),
        in_specs=[
            pl.BlockSpec((R, D), lambda i: (i, 0)),
            pl.BlockSpec((R, D), lambda i: (i, 0)),
            pl.BlockSpec((R, neg.shape[1]), lambda i: (i, 0)),
        ],
        out_specs=pl.BlockSpec((1, 1), lambda i: (0, 0)),
        out_shape=jax.ShapeDtypeStruct((1, 1), jnp.float32),
        interpret=_INTERPRET,
    )(z1f, z2f, neg)


def kernel(out1, out2, t):
    B, T, D = out1.shape
    z1, z2, tt = out1, out2, t.astype(jnp.float32)
    total = jnp.float32(0.0)
    d = 0
    while True:
        Tl = z1.shape[1]
        N = B * Tl
        tf = tt.reshape(N)
        z1f = z1.reshape(N, D)
        z2f = z2.reshape(N, D)
        idx = _topk_pallas(tf, Tl, B)
        neg = _gather_dot_sc(z1f, z2f, idx)
        total = total + _loss_pallas(z1f, z2f, neg)[0, 0] / N
        d += 1
        if Tl == 1:
            break
        T2 = Tl // 2
        tt = tt.reshape(B, T2, 2).mean(axis=2)
        z1 = z1.reshape(B, T2, 2, D).max(axis=2)
        z2 = z2.reshape(B, T2, 2, D).max(axis=2)
    return total / d


# speedup vs baseline: 10.5366x; 1.0012x over previous
import functools

import jax
import jax.numpy as jnp
from jax import lax
from jax.experimental import pallas as pl
from jax.experimental.pallas import tpu as pltpu
from jax.experimental.pallas import tpu_sc as plsc

_INTERPRET = False


def _topk_body(tc_ref, tr_ref, out_ref, *, T, B, R, N):
    i = pl.program_id(0)
    row_base = i * R
    trow = tc_ref[...]
    tall = tr_ref[...]
    diff = jnp.abs(trow - tall)
    col_iota = jax.lax.broadcasted_iota(jnp.int32, (R, N), 1)
    row_iota = jax.lax.broadcasted_iota(jnp.int32, (R, N), 0) + row_base
    same = (row_iota // T) == (col_iota // T)
    masked = jnp.where(same, jnp.float32(1e6), diff)
    key = (jax.lax.bitcast_convert_type(masked, jnp.int32) & ~0x1FFF) | col_iota
    cols = []
    for _ in range(B):
        m = jnp.min(key, axis=1, keepdims=True)
        cols.append(m & 0x1FFF)
        key = jnp.where(key == m, jnp.int32(0x7FFFFFFF), key)
    out_ref[...] = jnp.concatenate(cols, axis=1)


def _topk_pallas(tf, T, B):
    N = tf.shape[0]
    R = min(N, 256)
    t_col = tf.reshape(N, 1)
    t_row = tf.reshape(1, N)
    return pl.pallas_call(
        functools.partial(_topk_body, T=T, B=B, R=R, N=N),
        grid=(N // R,),
        in_specs=[
            pl.BlockSpec((R, 1), lambda i: (i, 0)),
            pl.BlockSpec((1, N), lambda i: (0, 0)),
        ],
        out_specs=pl.BlockSpec((R, B), lambda i: (i, 0)),
        out_shape=jax.ShapeDtypeStruct((N, B), jnp.int32),
        interpret=_INTERPRET,
    )(t_col, t_row)


def _gather_dot_sc(z1f, z2f, idx):
    N, D = z1f.shape
    B = idx.shape[1]
    info = plsc.get_sparse_core_info()
    NC, NS = info.num_cores, info.num_subcores
    NW = NC * NS
    RW = max(N // NW, 1)
    ACT = N // RW
    C = min(RW, 32)
    CHUNKS = RW // C
    idx_flat = idx.reshape(N * B)
    mesh = plsc.VectorSubcoreMesh(core_axis_name="c", subcore_axis_name="s")

    @functools.partial(
        pl.kernel, mesh=mesh,
        out_type=jax.ShapeDtypeStruct((N, 2 * B), jnp.float32),
        scratch_types=[
            pltpu.VMEM((C * B,), jnp.int32),
            pltpu.VMEM((C, D), jnp.float32),
            pltpu.VMEM((C * B, D), jnp.float32),
            pltpu.VMEM((C * B, D), jnp.float32),
            pltpu.VMEM((C, 2 * B), jnp.float32),
            pltpu.SemaphoreType.DMA,
        ])
    def sc_kernel(z1_hbm, z2_hbm, idx_hbm, out_hbm,
                  idx_v, own_v, g1_v, g2_v, out_v, sem):
        wid = lax.axis_index("s") * NC + lax.axis_index("c")

        @pl.when(wid < ACT)
        def _():
            def chunk_body(c, carry):
                rowbase = wid * RW + c * C
                pltpu.sync_copy(idx_hbm.at[pl.ds(rowbase * B, C * B)], idx_v)
                pltpu.async_copy(z1_hbm.at[idx_v], g1_v, sem).wait()
                pltpu.async_copy(z2_hbm.at[idx_v], g2_v, sem).wait()
                pltpu.sync_copy(z1_hbm.at[pl.ds(rowbase, C)], own_v)
                lane = lax.iota(jnp.int32, 16)
                perms = [((lane + s) & 15).reshape(16, 1) for s in (1, 2, 4, 8)]
                _gdims = lax.GatherDimensionNumbers(
                    offset_dims=(), collapsed_slice_dims=(0,),
                    start_index_map=(0,))

                def _lanesum(v):
                    for p in perms:
                        v = v + lax.gather(
                            v, p, _gdims, (1,),
                            mode=lax.GatherScatterMode.PROMISE_IN_BOUNDS)
                    return v

                def row_body(r, carry2):
                    z1r = [own_v[r, pl.ds(i * 16, 16)] for i in range(D // 16)]
                    vals = jnp.zeros((16,), jnp.float32)
                    for k in range(B):
                        acc1 = z1r[0] * g1_v[r * B + k, pl.ds(0, 16)]
                        acc2 = z1r[0] * g2_v[r * B + k, pl.ds(0, 16)]
                        for i in range(1, D // 16):
                            acc1 = acc1 + z1r[i] * g1_v[r * B + k, pl.ds(i * 16, 16)]
                            acc2 = acc2 + z1r[i] * g2_v[r * B + k, pl.ds(i * 16, 16)]
                        vals = jnp.where(lane == k, _lanesum(acc1), vals)
                        vals = jnp.where(lane == (k + B), _lanesum(acc2), vals)
                    out_v[r, :] = vals
                    return carry2

                lax.fori_loop(0, C, row_body, 0)
                pltpu.sync_copy(out_v, out_hbm.at[pl.ds(rowbase, C)])
                return carry

            lax.fori_loop(0, CHUNKS, chunk_body, 0)

    return sc_kernel(z1f, z2f, idx_flat)


def _loss_body(z1_ref, z2_ref, neg_ref, out_ref):
    i = pl.program_id(0)
    z1 = z1_ref[...]
    z2 = z2_ref[...]
    neg = neg_ref[...]
    pos = jnp.sum(z1 * z2, axis=1, keepdims=True)
    m = jnp.maximum(jnp.max(neg, axis=1, keepdims=True), pos)
    s = jnp.sum(jnp.exp(neg - m), axis=1, keepdims=True) + jnp.exp(pos - m)
    lse = m + jnp.log(s)
    part = jnp.sum(lse - pos).reshape(1, 1)

    @pl.when(i == 0)
    def _():
        out_ref[...] = jnp.zeros((1, 1), jnp.float32)

    out_ref[...] += part


def _loss_pallas(z1f, z2f, neg):
    N, D = z1f.shape
    R = min(N, 256)
    return pl.pallas_call(
        _loss_body,
        grid=(N // R,),
        in_specs=[
            pl.BlockSpec((R, D), lambda i: (i, 0)),
            pl.BlockSpec((R, D), lambda i: (i, 0)),
            pl.BlockSpec((R, neg.shape[1]), lambda i: (i, 0)),
        ],
        out_specs=pl.BlockSpec((1, 1), lambda i: (0, 0)),
        out_shape=jax.ShapeDtypeStruct((1, 1), jnp.float32),
        interpret=_INTERPRET,
    )(z1f, z2f, neg)


def kernel(out1, out2, t):
    B, T, D = out1.shape
    z1, z2, tt = out1, out2, t.astype(jnp.float32)
    levels = []
    while True:
        Tl = z1.shape[1]
        N = B * Tl
        levels.append((z1.reshape(N, D), z2.reshape(N, D), tt.reshape(N), Tl))
        if Tl == 1:
            break
        T2 = Tl // 2
        tt = tt.reshape(B, T2, 2).mean(axis=2)
        z1 = z1.reshape(B, T2, 2, D).max(axis=2)
        z2 = z2.reshape(B, T2, 2, D).max(axis=2)
    idxs = [_topk_pallas(tf, Tl, B) for (_, _, tf, Tl) in levels]
    negs = [_gather_dot_sc(z1f, z2f, idx)
            for (z1f, z2f, _, _), idx in zip(levels, idxs)]
    total = jnp.float32(0.0)
    for (z1f, z2f, _, Tl), neg in zip(levels, negs):
        total = total + _loss_pallas(z1f, z2f, neg)[0, 0] / (B * Tl)
    return total / len(levels)


# speedup vs baseline: 13.2324x; 1.2558x over previous
import functools

import jax
import jax.numpy as jnp
from jax import lax
from jax.experimental import pallas as pl
from jax.experimental.pallas import tpu as pltpu
from jax.experimental.pallas import tpu_sc as plsc

_INTERPRET = False


def _topk_body(tc_ref, tr_ref, out_ref, *, T, B, R, N):
    i = pl.program_id(0)
    row_base = i * R
    trow = tc_ref[...]
    tall = tr_ref[...]
    diff = jnp.abs(trow - tall)
    col_iota = jax.lax.broadcasted_iota(jnp.int32, (R, N), 1)
    row_iota = jax.lax.broadcasted_iota(jnp.int32, (R, N), 0) + row_base
    same = (row_iota // T) == (col_iota // T)
    masked = jnp.where(same, jnp.float32(1e6), diff)
    key = jax.lax.bitcast_convert_type(
        (jax.lax.bitcast_convert_type(masked, jnp.int32) & ~0x1FFF) | col_iota,
        jnp.float32)
    cols = []
    for _ in range(B):
        m = jnp.min(key, axis=1, keepdims=True)
        cols.append(jax.lax.bitcast_convert_type(m, jnp.int32) & 0x1FFF)
        key = jnp.where(key == m, jnp.float32(jnp.inf), key)
    out_ref[...] = jnp.concatenate(cols, axis=1)


def _topk_pallas(tf, T, B):
    N = tf.shape[0]
    R = min(N, 256)
    t_col = tf.reshape(N, 1)
    t_row = tf.reshape(1, N)
    return pl.pallas_call(
        functools.partial(_topk_body, T=T, B=B, R=R, N=N),
        grid=(N // R,),
        in_specs=[
            pl.BlockSpec((R, 1), lambda i: (i, 0)),
            pl.BlockSpec((1, N), lambda i: (0, 0)),
        ],
        out_specs=pl.BlockSpec((R, B), lambda i: (i, 0)),
        out_shape=jax.ShapeDtypeStruct((N, B), jnp.int32),
        interpret=_INTERPRET,
    )(t_col, t_row)


def _gather_dot_sc(z1f, z2f, idx):
    N, D = z1f.shape
    B = idx.shape[1]
    info = plsc.get_sparse_core_info()
    NC, NS = info.num_cores, info.num_subcores
    NW = NC * NS
    RW = max(N // NW, 1)
    ACT = N // RW
    C = min(RW, 32)
    CHUNKS = RW // C
    idx_flat = idx.reshape(N * B)
    mesh = plsc.VectorSubcoreMesh(core_axis_name="c", subcore_axis_name="s")

    @functools.partial(
        pl.kernel, mesh=mesh,
        out_type=jax.ShapeDtypeStruct((N, 2 * B), jnp.float32),
        scratch_types=[
            pltpu.VMEM((C * B,), jnp.int32),
            pltpu.VMEM((C, D), jnp.float32),
            pltpu.VMEM((C * B, D), jnp.float32),
            pltpu.VMEM((C * B, D), jnp.float32),
            pltpu.VMEM((C, 2 * B), jnp.float32),
            pltpu.SemaphoreType.DMA,
        ])
    def sc_kernel(z1_hbm, z2_hbm, idx_hbm, out_hbm,
                  idx_v, own_v, g1_v, g2_v, out_v, sem):
        wid = lax.axis_index("s") * NC + lax.axis_index("c")

        @pl.when(wid < ACT)
        def _():
            def chunk_body(c, carry):
                rowbase = wid * RW + c * C
                pltpu.sync_copy(idx_hbm.at[pl.ds(rowbase * B, C * B)], idx_v)
                pltpu.async_copy(z1_hbm.at[idx_v], g1_v, sem).wait()
                pltpu.async_copy(z2_hbm.at[idx_v], g2_v, sem).wait()
                pltpu.sync_copy(z1_hbm.at[pl.ds(rowbase, C)], own_v)
                lane = lax.iota(jnp.int32, 16)
                perms = [((lane + s) & 15).reshape(16, 1) for s in (1, 2, 4, 8)]
                _gdims = lax.GatherDimensionNumbers(
                    offset_dims=(), collapsed_slice_dims=(0,),
                    start_index_map=(0,))

                def _lanesum(v):
                    for p in perms:
                        v = v + lax.gather(
                            v, p, _gdims, (1,),
                            mode=lax.GatherScatterMode.PROMISE_IN_BOUNDS)
                    return v

                def row_body(r, carry2):
                    z1r = [own_v[r, pl.ds(i * 16, 16)] for i in range(D // 16)]
                    vals = jnp.zeros((16,), jnp.float32)
                    for k in range(B):
                        acc1 = z1r[0] * g1_v[r * B + k, pl.ds(0, 16)]
                        acc2 = z1r[0] * g2_v[r * B + k, pl.ds(0, 16)]
                        for i in range(1, D // 16):
                            acc1 = acc1 + z1r[i] * g1_v[r * B + k, pl.ds(i * 16, 16)]
                            acc2 = acc2 + z1r[i] * g2_v[r * B + k, pl.ds(i * 16, 16)]
                        vals = jnp.where(lane == k, _lanesum(acc1), vals)
                        vals = jnp.where(lane == (k + B), _lanesum(acc2), vals)
                    out_v[r, :] = vals
                    return carry2

                lax.fori_loop(0, C, row_body, 0)
                pltpu.sync_copy(out_v, out_hbm.at[pl.ds(rowbase, C)])
                return carry

            lax.fori_loop(0, CHUNKS, chunk_body, 0)

    return sc_kernel(z1f, z2f, idx_flat)


def _loss_body(z1_ref, z2_ref, neg_ref, out_ref):
    i = pl.program_id(0)
    z1 = z1_ref[...]
    z2 = z2_ref[...]
    neg = neg_ref[...]
    pos = jnp.sum(z1 * z2, axis=1, keepdims=True)
    m = jnp.maximum(jnp.max(neg, axis=1, keepdims=True), pos)
    s = jnp.sum(jnp.exp(neg - m), axis=1, keepdims=True) + jnp.exp(pos - m)
    lse = m + jnp.log(s)
    part = jnp.sum(lse - pos).reshape(1, 1)

    @pl.when(i == 0)
    def _():
        out_ref[...] = jnp.zeros((1, 1), jnp.float32)

    out_ref[...] += part


def _loss_pallas(z1f, z2f, neg):
    N, D = z1f.shape
    R = min(N, 256)
    return pl.pallas_call(
        _loss_body,
        grid=(N // R,),
        in_specs=[
            pl.BlockSpec((R, D), lambda i: (i, 0)),
            pl.BlockSpec((R, D), lambda i: (i, 0)),
            pl.BlockSpec((R, neg.shape[1]), lambda i: (i, 0)),
        ],
        out_specs=pl.BlockSpec((1, 1), lambda i: (0, 0)),
        out_shape=jax.ShapeDtypeStruct((1, 1), jnp.float32),
        interpret=_INTERPRET,
    )(z1f, z2f, neg)


def kernel(out1, out2, t):
    B, T, D = out1.shape
    z1, z2, tt = out1, out2, t.astype(jnp.float32)
    levels = []
    while True:
        Tl = z1.shape[1]
        N = B * Tl
        levels.append((z1.reshape(N, D), z2.reshape(N, D), tt.reshape(N), Tl))
        if Tl == 1:
            break
        T2 = Tl // 2
        tt = tt.reshape(B, T2, 2).mean(axis=2)
        z1 = z1.reshape(B, T2, 2, D).max(axis=2)
        z2 = z2.reshape(B, T2, 2, D).max(axis=2)
    idxs = [_topk_pallas(tf, Tl, B) for (_, _, tf, Tl) in levels]
    negs = [_gather_dot_sc(z1f, z2f, idx)
            for (z1f, z2f, _, _), idx in zip(levels, idxs)]
    total = jnp.float32(0.0)
    for (z1f, z2f, _, Tl), neg in zip(levels, negs):
        total = total + _loss_pallas(z1f, z2f, neg)[0, 0] / (B * Tl)
    return total / len(levels)
